# baseline (speedup 1.0000x reference)
import numpy as np

import jax
import jax.numpy as jnp
from jax import lax
from jax.experimental import pallas as pl
from jax.experimental.pallas import tpu as pltpu
from jax.experimental.pallas import tpu_sc as plsc

B = 16384
H = 50
D = 64
NC, NS = 2, 16
NW = NC * NS
EPW = B // NW
IPW = EPW * H
CE = 4
CPW = CE * H
NCH = EPW // CE
SPLIT = (40, 40, 40, 40, 40)
NB = 4
RU = 10

_half = np.arange(16, dtype=np.int32)
_inter = np.empty(32, dtype=np.int32)
_inter[0::2] = _half
_inter[1::2] = _half + 16
_COLPERM = np.concatenate([_inter, _inter + 32])


def _body(x_ref, w_ref, b_ref, o_ref, idx_v, bias_v, out_v,
          rows0, rows1, rows2, rows3, sem0, sem1, sem2, sem3):
    rows = (rows0, rows1, rows2, rows3)
    sems = (sem0, sem1, sem2, sem3)
    wid = lax.axis_index("s") * NC + lax.axis_index("c")

    pltpu.sync_copy(x_ref.at[wid], idx_v)
    pltpu.sync_copy(b_ref, bias_v)
    bias_vec = [bias_v[pl.ds(k * 16, 16)] for k in range(4)]
    inv_h = jnp.float32(1.0 / H)

    def start_gathers(c, b):
        off = 0
        for n in SPLIT:
            pltpu.async_copy(
                w_ref.at[idx_v.at[pl.ds(c * CPW + off, n)]],
                rows[b].at[pl.ds(off, n)], sems[b])
            off += n

    def wait_gathers(c, b):
        off = 0
        for n in SPLIT:
            pltpu.make_async_copy(
                w_ref.at[idx_v.at[pl.ds(c * CPW + off, n)]],
                rows[b].at[pl.ds(off, n)], sems[b]).wait()
            off += n

    for b in range(NB):
        start_gathers(b, b)

    @pl.loop(0, NCH, step=NB)
    def _chunks(j):
        for b in range(NB):
            c = j + b
            wait_gathers(c, b)
            for e in range(CE):
                base = e * H

                def rbody(it, acc, _b=b, _base=base):
                    r0 = _base + it * RU
                    a = list(acc)
                    for u in range(RU):
                        for h in range(2):
                            v = rows[_b][r0 + u, pl.ds(h * 32, 32)]
                            lo, hi = plsc.unpack(
                                v, format=plsc.PackFormat.INTERLEAVED)
                            a[2 * h] = a[2 * h] + lo
                            a[2 * h + 1] = a[2 * h + 1] + hi
                    return tuple(a)

                z = jnp.zeros((16,), jnp.float32)
                acc = lax.fori_loop(0, H // RU, rbody, (z, z, z, z))
                orow = c * CE + e
                for k in range(4):
                    out_v[orow, pl.ds(k * 16, 16)] = (
                        acc[k] * inv_h + bias_vec[k])

            @pl.when(c + NB < NCH)
            def _():
                start_gathers(c + NB, b)

    pltpu.sync_copy(out_v, o_ref.at[pl.ds(wid * EPW, EPW)])


@jax.jit
def _emb_sum(x3, wbf16, emb_bias):
    mesh = plsc.VectorSubcoreMesh(core_axis_name="c", subcore_axis_name="s")
    f = pl.kernel(
        _body,
        out_type=jax.ShapeDtypeStruct((B, D), jnp.float32),
        mesh=mesh,
        scratch_types=[
            pltpu.VMEM((IPW,), jnp.int32),
            pltpu.VMEM((D,), jnp.float32),
            pltpu.VMEM((EPW, D), jnp.float32),
        ] + [pltpu.VMEM((CPW, D), jnp.bfloat16) for _ in range(NB)]
          + [pltpu.SemaphoreType.DMA for _ in range(NB)],
        compiler_params=pltpu.CompilerParams(
            use_tc_tiling_on_sc=False, needs_layout_passes=False),
    )
    return f(x3, wbf16, emb_bias)


def kernel(x, weight, emb_bias):
    x3 = x.astype(jnp.int32).reshape(NW, IPW)
    wbf16 = weight[:, _COLPERM].astype(jnp.bfloat16)
    return _emb_sum(x3, wbf16, emb_bias)
